# Initial kernel scaffold; baseline (speedup 1.0000x reference)
import functools

import jax
import jax.numpy as jnp
from jax import lax
from jax.experimental import pallas as pl
from jax.experimental.pallas import tpu as pltpu
from jax.experimental.pallas import tpu_sc as plsc

N = 10000
E = 320000
D_IN = 128
D_HID = 64

NC, NS = 2, 16
NW = NC * NS
CHUNK = 128
JPT = 79
EPT = JPT * CHUNK
EPAD = NW * EPT
ACC_ROWS = 10240
TRASH = N
RPT = N // NS
ZPT = ACC_ROWS // NS

_MESH = plsc.VectorSubcoreMesh(core_axis_name="c", subcore_axis_name="s")


@functools.partial(
    pl.kernel,
    out_type=jax.ShapeDtypeStruct((NC, ACC_ROWS), jnp.float32),
    mesh=_MESH,
    scratch_types=[
        pltpu.VMEM_SHARED((ACC_ROWS,), jnp.float32),
        pltpu.VMEM((JPT, CHUNK), jnp.int32),
        pltpu.VMEM((CHUNK,), jnp.float32),
        pltpu.VMEM((ZPT,), jnp.float32),
    ],
)
def _deg_kernel(dst_hbm, out_hbm, acc, dst_idx, ones_v, zeros_v):
    cid = lax.axis_index("c")
    sid = lax.axis_index("s")
    wid = cid * NS + sid

    for i in range(ZPT // 16):
        zeros_v[pl.ds(16 * i, 16)] = jnp.zeros((16,), jnp.float32)
    for i in range(CHUNK // 16):
        ones_v[pl.ds(16 * i, 16)] = jnp.ones((16,), jnp.float32)

    pltpu.sync_copy(dst_hbm.at[pl.ds(wid * JPT, JPT)], dst_idx)
    pltpu.sync_copy(zeros_v, acc.at[pl.ds(sid * ZPT, ZPT)])
    plsc.subcore_barrier()

    def body(j, carry):
        pltpu.sync_copy(ones_v, acc.at[dst_idx.at[j]], add=True)
        return carry

    lax.fori_loop(0, JPT, body, 0)
    plsc.subcore_barrier()
    pltpu.sync_copy(acc.at[pl.ds(sid * ZPT, ZPT)],
                    out_hbm.at[cid, pl.ds(sid * ZPT, ZPT)])


@functools.partial(
    pl.kernel,
    out_type=jax.ShapeDtypeStruct((NC, N, D_HID), jnp.float32),
    mesh=_MESH,
    scratch_types=[
        pltpu.VMEM_SHARED((ACC_ROWS, D_HID), jnp.float32),
        pltpu.VMEM((JPT, CHUNK), jnp.int32),
        pltpu.VMEM((JPT, CHUNK), jnp.int32),
        pltpu.VMEM((CHUNK, D_HID), jnp.float32),
        pltpu.SemaphoreType.DMA,
    ],
)
def _agg_kernel(y_hbm, src_hbm, dst_hbm, out_hbm, acc, src_idx, dst_idx, rows,
                sem):
    cid = lax.axis_index("c")
    sid = lax.axis_index("s")
    wid = cid * NS + sid

    pltpu.sync_copy(src_hbm.at[pl.ds(wid * JPT, JPT)], src_idx)
    pltpu.sync_copy(dst_hbm.at[pl.ds(wid * JPT, JPT)], dst_idx)
    r0 = sid * RPT
    pltpu.sync_copy(y_hbm.at[pl.ds(r0, RPT)], acc.at[pl.ds(r0, RPT)])
    plsc.subcore_barrier()

    def body(j, carry):
        pltpu.async_copy(y_hbm.at[src_idx.at[j]], rows, sem).wait()
        pltpu.sync_copy(rows, acc.at[dst_idx.at[j]], add=True)
        return carry

    lax.fori_loop(0, JPT, body, 0)
    plsc.subcore_barrier()
    pltpu.sync_copy(acc.at[pl.ds(r0, RPT)], out_hbm.at[cid, pl.ds(r0, RPT)])


_BS = 1000


def _tc_scale_matmul(x_ref, w_ref, da_ref, db_ref, y_ref, dinv_ref):
    deg = 1.0 + da_ref[...] + db_ref[...]
    dinv = lax.rsqrt(deg)
    y_ref[...] = dinv * jnp.dot(x_ref[...], w_ref[...],
                                preferred_element_type=jnp.float32)
    dinv_ref[...] = dinv


def _tc_mid(a_ref, b_ref, y_ref, dinv_ref, bias_ref, w_ref, out_ref):
    s = a_ref[...] + b_ref[...] - y_ref[...]
    h = jnp.maximum(dinv_ref[...] * s + bias_ref[...], 0.0)
    out_ref[...] = dinv_ref[...] * jnp.dot(h, w_ref[...],
                                           preferred_element_type=jnp.float32)


def _tc_out(a_ref, b_ref, y_ref, dinv_ref, bias_ref, w_ref, bout_ref, out_ref):
    s = a_ref[...] + b_ref[...] - y_ref[...]
    h = jnp.maximum(dinv_ref[...] * s + bias_ref[...], 0.0)
    out_ref[...] = jnp.dot(h, w_ref[...],
                           preferred_element_type=jnp.float32) + bout_ref[...]


def _row_spec(d):
    return pl.BlockSpec((_BS, d), lambda i: (i, 0))


def _full_spec(r, c):
    return pl.BlockSpec((r, c), lambda i: (0, 0))


_scale_matmul = pl.pallas_call(
    _tc_scale_matmul,
    grid=(N // _BS,),
    in_specs=[_row_spec(D_IN), _full_spec(D_IN, D_HID), _row_spec(1),
              _row_spec(1)],
    out_specs=[_row_spec(D_HID), _row_spec(1)],
    out_shape=[jax.ShapeDtypeStruct((N, D_HID), jnp.float32),
               jax.ShapeDtypeStruct((N, 1), jnp.float32)],
)

_mid = pl.pallas_call(
    _tc_mid,
    grid=(N // _BS,),
    in_specs=[_row_spec(D_HID), _row_spec(D_HID), _row_spec(D_HID),
              _row_spec(1), _full_spec(1, D_HID), _full_spec(D_HID, D_HID)],
    out_specs=_row_spec(D_HID),
    out_shape=jax.ShapeDtypeStruct((N, D_HID), jnp.float32),
)

_out = pl.pallas_call(
    _tc_out,
    grid=(N // _BS,),
    in_specs=[_row_spec(D_HID), _row_spec(D_HID), _row_spec(D_HID),
              _row_spec(1), _full_spec(1, D_HID), _full_spec(D_HID, 1),
              _full_spec(1, 1)],
    out_specs=_row_spec(1),
    out_shape=jax.ShapeDtypeStruct((N, 1), jnp.float32),
)


def kernel(x, edge_index, W1, b1, W2, b2, Wout, bout):
    src = edge_index[0]
    dst = edge_index[1]
    pad = EPAD - E
    srcp = jnp.concatenate(
        [src, jnp.zeros((pad,), jnp.int32)]).reshape(NW * JPT, CHUNK)
    dstp = jnp.concatenate(
        [dst, jnp.full((pad,), TRASH, jnp.int32)]).reshape(NW * JPT, CHUNK)

    degp = _deg_kernel(dstp)
    dega = degp[0, :N].reshape(N, 1)
    degb = degp[1, :N].reshape(N, 1)

    y1, dinv = _scale_matmul(x, W1, dega, degb)
    acc1 = _agg_kernel(y1, srcp, dstp)
    y2 = _mid(acc1[0], acc1[1], y1, dinv, b1.reshape(1, D_HID), W2)
    acc2 = _agg_kernel(y2, srcp, dstp)
    out = _out(acc2[0], acc2[1], y2, dinv, b2.reshape(1, D_HID), Wout,
               bout.reshape(1, 1))
    return out.reshape(N)


# speedup vs baseline: 16.1145x; 16.1145x over previous
import functools

import jax
import jax.numpy as jnp
from jax import lax
from jax.experimental import pallas as pl
from jax.experimental.pallas import tpu as pltpu
from jax.experimental.pallas import tpu_sc as plsc

N = 10000
NP = 10240
E = 320000
D_IN = 128
D_HID = 64

NC, NS = 2, 16
NW = NC * NS
CHUNK = 128
JPT = 80
EPT = JPT * CHUNK
EPAD = NW * EPT
TRASH = N
RPT = NP // NS

_MESH = plsc.VectorSubcoreMesh(core_axis_name="c", subcore_axis_name="s")


@functools.partial(
    pl.kernel,
    out_type=jax.ShapeDtypeStruct((NC, NP), jnp.float32),
    mesh=_MESH,
    scratch_types=[
        pltpu.VMEM_SHARED((NP,), jnp.float32),
        pltpu.VMEM((JPT, CHUNK), jnp.int32),
        pltpu.VMEM((CHUNK,), jnp.float32),
        pltpu.VMEM((RPT,), jnp.float32),
    ],
    compiler_params=pltpu.CompilerParams(use_tc_tiling_on_sc=False),
)
def _deg_kernel(dst_hbm, out_hbm, acc, dst_idx, ones_v, zeros_v):
    cid = lax.axis_index("c")
    sid = lax.axis_index("s")
    wid = cid * NS + sid

    for i in range(RPT // 16):
        zeros_v[pl.ds(16 * i, 16)] = jnp.zeros((16,), jnp.float32)
    for i in range(CHUNK // 16):
        ones_v[pl.ds(16 * i, 16)] = jnp.ones((16,), jnp.float32)

    pltpu.sync_copy(dst_hbm.at[pl.ds(wid * JPT, JPT)], dst_idx)
    pltpu.sync_copy(zeros_v, acc.at[pl.ds(sid * RPT, RPT)])
    plsc.subcore_barrier()

    def body(j, carry):
        pltpu.sync_copy(ones_v, acc.at[dst_idx.at[j]], add=True)
        return carry

    lax.fori_loop(0, JPT, body, 0)
    plsc.subcore_barrier()
    pltpu.sync_copy(acc.at[pl.ds(sid * RPT, RPT)],
                    out_hbm.at[cid, pl.ds(sid * RPT, RPT)])


@functools.partial(
    pl.kernel,
    out_type=jax.ShapeDtypeStruct((NC, NP, D_HID), jnp.float32),
    mesh=_MESH,
    scratch_types=[
        pltpu.VMEM_SHARED((NP, D_HID), jnp.float32),
        pltpu.VMEM((JPT, CHUNK), jnp.int32),
        pltpu.VMEM((JPT, CHUNK), jnp.int32),
        pltpu.VMEM((CHUNK, D_HID), jnp.float32),
        pltpu.SemaphoreType.DMA,
    ],
    compiler_params=pltpu.CompilerParams(use_tc_tiling_on_sc=False),
)
def _agg_kernel(y_hbm, src_hbm, dst_hbm, out_hbm, acc, src_idx, dst_idx, rows,
                sem):
    cid = lax.axis_index("c")
    sid = lax.axis_index("s")
    wid = cid * NS + sid

    pltpu.sync_copy(src_hbm.at[pl.ds(wid * JPT, JPT)], src_idx)
    pltpu.sync_copy(dst_hbm.at[pl.ds(wid * JPT, JPT)], dst_idx)
    r0 = sid * RPT
    pltpu.sync_copy(y_hbm.at[pl.ds(r0, RPT)], acc.at[pl.ds(r0, RPT)])
    plsc.subcore_barrier()

    def body(j, carry):
        pltpu.async_copy(y_hbm.at[src_idx.at[j]], rows, sem).wait()
        pltpu.sync_copy(rows, acc.at[dst_idx.at[j]], add=True)
        return carry

    lax.fori_loop(0, JPT, body, 0)
    plsc.subcore_barrier()
    pltpu.sync_copy(acc.at[pl.ds(r0, RPT)], out_hbm.at[cid, pl.ds(r0, RPT)])


_BS = 1024


def _tc_scale_matmul(x_ref, w_ref, da_ref, db_ref, y_ref, dinv_ref):
    deg = 1.0 + da_ref[...] + db_ref[...]
    dinv = lax.rsqrt(deg)
    y_ref[...] = dinv * jnp.dot(x_ref[...], w_ref[...],
                                preferred_element_type=jnp.float32)
    dinv_ref[...] = dinv


def _tc_mid(a_ref, b_ref, y_ref, dinv_ref, bias_ref, w_ref, out_ref):
    s = a_ref[...] + b_ref[...] - y_ref[...]
    h = jnp.maximum(dinv_ref[...] * s + bias_ref[...], 0.0)
    out_ref[...] = dinv_ref[...] * jnp.dot(h, w_ref[...],
                                           preferred_element_type=jnp.float32)


def _tc_out(a_ref, b_ref, y_ref, dinv_ref, bias_ref, w_ref, bout_ref, out_ref):
    s = a_ref[...] + b_ref[...] - y_ref[...]
    h = jnp.maximum(dinv_ref[...] * s + bias_ref[...], 0.0)
    out_ref[...] = jnp.dot(h, w_ref[...],
                           preferred_element_type=jnp.float32) + bout_ref[...]


def _row_spec(d):
    return pl.BlockSpec((_BS, d), lambda i: (i, 0))


def _full_spec(r, c):
    return pl.BlockSpec((r, c), lambda i: (0, 0))


_scale_matmul = pl.pallas_call(
    _tc_scale_matmul,
    grid=(NP // _BS,),
    in_specs=[_row_spec(D_IN), _full_spec(D_IN, D_HID), _row_spec(1),
              _row_spec(1)],
    out_specs=[_row_spec(D_HID), _row_spec(1)],
    out_shape=[jax.ShapeDtypeStruct((NP, D_HID), jnp.float32),
               jax.ShapeDtypeStruct((NP, 1), jnp.float32)],
)

_mid = pl.pallas_call(
    _tc_mid,
    grid=(NP // _BS,),
    in_specs=[_row_spec(D_HID), _row_spec(D_HID), _row_spec(D_HID),
              _row_spec(1), _full_spec(1, D_HID), _full_spec(D_HID, D_HID)],
    out_specs=_row_spec(D_HID),
    out_shape=jax.ShapeDtypeStruct((NP, D_HID), jnp.float32),
)

_out = pl.pallas_call(
    _tc_out,
    grid=(NP // _BS,),
    in_specs=[_row_spec(D_HID), _row_spec(D_HID), _row_spec(D_HID),
              _row_spec(1), _full_spec(1, D_HID), _full_spec(D_HID, 1),
              _full_spec(1, 1)],
    out_specs=_row_spec(1),
    out_shape=jax.ShapeDtypeStruct((NP, 1), jnp.float32),
)


def kernel(x, edge_index, W1, b1, W2, b2, Wout, bout):
    src = edge_index[0]
    dst = edge_index[1]
    pad = EPAD - E
    srcp = jnp.concatenate(
        [src, jnp.zeros((pad,), jnp.int32)]).reshape(NW * JPT, CHUNK)
    dstp = jnp.concatenate(
        [dst, jnp.full((pad,), TRASH, jnp.int32)]).reshape(NW * JPT, CHUNK)
    xp = jnp.pad(x, ((0, NP - N), (0, 0)))

    degp = _deg_kernel(dstp)
    dega = degp[0].reshape(NP, 1)
    degb = degp[1].reshape(NP, 1)

    y1, dinv = _scale_matmul(xp, W1, dega, degb)
    acc1 = _agg_kernel(y1, srcp, dstp)
    y2 = _mid(acc1[0], acc1[1], y1, dinv, b1.reshape(1, D_HID), W2)
    acc2 = _agg_kernel(y2, srcp, dstp)
    out = _out(acc2[0], acc2[1], y2, dinv, b2.reshape(1, D_HID), Wout,
               bout.reshape(1, 1))
    return out.reshape(NP)[:N]


# speedup vs baseline: 19.1460x; 1.1881x over previous
import functools

import jax
import jax.numpy as jnp
from jax import lax
from jax.experimental import pallas as pl
from jax.experimental.pallas import tpu as pltpu
from jax.experimental.pallas import tpu_sc as plsc

N = 10000
NP = 10240
E = 320000
D_IN = 128
D_HID = 64

NC, NS = 2, 16
NW = NC * NS
CHUNK = 128
JPT = 80
EPT = JPT * CHUNK
EPAD = NW * EPT
TRASH = N
RPT = NP // NS

_MESH = plsc.VectorSubcoreMesh(core_axis_name="c", subcore_axis_name="s")


@functools.partial(
    pl.kernel,
    out_type=jax.ShapeDtypeStruct((NC, NP), jnp.float32),
    mesh=_MESH,
    scratch_types=[
        pltpu.VMEM_SHARED((NP,), jnp.float32),
        pltpu.VMEM((JPT, CHUNK), jnp.int32),
        pltpu.VMEM((CHUNK,), jnp.float32),
        pltpu.VMEM((RPT,), jnp.float32),
    ],
    compiler_params=pltpu.CompilerParams(use_tc_tiling_on_sc=False),
)
def _deg_kernel(dst_hbm, out_hbm, acc, dst_idx, ones_v, zeros_v):
    cid = lax.axis_index("c")
    sid = lax.axis_index("s")
    wid = cid * NS + sid

    for i in range(RPT // 16):
        zeros_v[pl.ds(16 * i, 16)] = jnp.zeros((16,), jnp.float32)
    for i in range(CHUNK // 16):
        ones_v[pl.ds(16 * i, 16)] = jnp.ones((16,), jnp.float32)

    pltpu.sync_copy(dst_hbm.at[pl.ds(wid * JPT, JPT)], dst_idx)
    pltpu.sync_copy(zeros_v, acc.at[pl.ds(sid * RPT, RPT)])
    plsc.subcore_barrier()

    def body(j, carry):
        pltpu.sync_copy(ones_v, acc.at[dst_idx.at[j]], add=True)
        return carry

    lax.fori_loop(0, JPT, body, 0)
    plsc.subcore_barrier()
    pltpu.sync_copy(acc.at[pl.ds(sid * RPT, RPT)],
                    out_hbm.at[cid, pl.ds(sid * RPT, RPT)])


@functools.partial(
    pl.kernel,
    out_type=jax.ShapeDtypeStruct((NC, NP, D_HID), jnp.float32),
    mesh=_MESH,
    scratch_types=[
        pltpu.VMEM_SHARED((NP, D_HID), jnp.float32),
        pltpu.VMEM((JPT, CHUNK), jnp.int32),
        pltpu.VMEM((JPT, CHUNK), jnp.int32),
        pltpu.VMEM((CHUNK, D_HID), jnp.float32),
        pltpu.VMEM((CHUNK, D_HID), jnp.float32),
        pltpu.SemaphoreType.DMA,
        pltpu.SemaphoreType.DMA,
    ],
    compiler_params=pltpu.CompilerParams(use_tc_tiling_on_sc=False),
)
def _agg_kernel(y_hbm, src_hbm, dst_hbm, out_hbm, acc, src_idx, dst_idx,
                rows0, rows1, sem0, sem1):
    cid = lax.axis_index("c")
    sid = lax.axis_index("s")
    wid = cid * NS + sid

    pltpu.sync_copy(src_hbm.at[pl.ds(wid * JPT, JPT)], src_idx)
    pltpu.sync_copy(dst_hbm.at[pl.ds(wid * JPT, JPT)], dst_idx)
    r0 = sid * RPT
    pltpu.sync_copy(y_hbm.at[pl.ds(r0, RPT)], acc.at[pl.ds(r0, RPT)])
    plsc.subcore_barrier()

    pltpu.async_copy(y_hbm.at[src_idx.at[0]], rows0, sem0)
    pltpu.async_copy(y_hbm.at[src_idx.at[1]], rows1, sem1)

    def half(j, rows, sem):
        pltpu.make_async_copy(y_hbm.at[src_idx.at[j]], rows, sem).wait()
        pltpu.sync_copy(rows, acc.at[dst_idx.at[j]], add=True)

        @pl.when(j + 2 < JPT)
        def _():
            pltpu.async_copy(y_hbm.at[src_idx.at[j + 2]], rows, sem)

    def body(jj, carry):
        half(2 * jj, rows0, sem0)
        half(2 * jj + 1, rows1, sem1)
        return carry

    lax.fori_loop(0, JPT // 2, body, 0)
    plsc.subcore_barrier()
    pltpu.sync_copy(acc.at[pl.ds(r0, RPT)], out_hbm.at[cid, pl.ds(r0, RPT)])


_BS = 1024


def _tc_scale_matmul(x_ref, w_ref, da_ref, db_ref, y_ref, dinv_ref):
    deg = 1.0 + da_ref[...] + db_ref[...]
    dinv = lax.rsqrt(deg)
    y_ref[...] = dinv * jnp.dot(x_ref[...], w_ref[...],
                                preferred_element_type=jnp.float32)
    dinv_ref[...] = dinv


def _tc_mid(a_ref, b_ref, y_ref, dinv_ref, bias_ref, w_ref, out_ref):
    s = a_ref[...] + b_ref[...] - y_ref[...]
    h = jnp.maximum(dinv_ref[...] * s + bias_ref[...], 0.0)
    out_ref[...] = dinv_ref[...] * jnp.dot(h, w_ref[...],
                                           preferred_element_type=jnp.float32)


def _tc_out(a_ref, b_ref, y_ref, dinv_ref, bias_ref, w_ref, bout_ref, out_ref):
    s = a_ref[...] + b_ref[...] - y_ref[...]
    h = jnp.maximum(dinv_ref[...] * s + bias_ref[...], 0.0)
    out_ref[...] = jnp.dot(h, w_ref[...],
                           preferred_element_type=jnp.float32) + bout_ref[...]


def _row_spec(d):
    return pl.BlockSpec((_BS, d), lambda i: (i, 0))


def _full_spec(r, c):
    return pl.BlockSpec((r, c), lambda i: (0, 0))


_scale_matmul = pl.pallas_call(
    _tc_scale_matmul,
    grid=(NP // _BS,),
    in_specs=[_row_spec(D_IN), _full_spec(D_IN, D_HID), _row_spec(1),
              _row_spec(1)],
    out_specs=[_row_spec(D_HID), _row_spec(1)],
    out_shape=[jax.ShapeDtypeStruct((NP, D_HID), jnp.float32),
               jax.ShapeDtypeStruct((NP, 1), jnp.float32)],
)

_mid = pl.pallas_call(
    _tc_mid,
    grid=(NP // _BS,),
    in_specs=[_row_spec(D_HID), _row_spec(D_HID), _row_spec(D_HID),
              _row_spec(1), _full_spec(1, D_HID), _full_spec(D_HID, D_HID)],
    out_specs=_row_spec(D_HID),
    out_shape=jax.ShapeDtypeStruct((NP, D_HID), jnp.float32),
)

_out = pl.pallas_call(
    _tc_out,
    grid=(NP // _BS,),
    in_specs=[_row_spec(D_HID), _row_spec(D_HID), _row_spec(D_HID),
              _row_spec(1), _full_spec(1, D_HID), _full_spec(D_HID, 1),
              _full_spec(1, 1)],
    out_specs=_row_spec(1),
    out_shape=jax.ShapeDtypeStruct((NP, 1), jnp.float32),
)


def kernel(x, edge_index, W1, b1, W2, b2, Wout, bout):
    src = edge_index[0]
    dst = edge_index[1]
    pad = EPAD - E
    srcp = jnp.concatenate(
        [src, jnp.zeros((pad,), jnp.int32)]).reshape(NW * JPT, CHUNK)
    dstp = jnp.concatenate(
        [dst, jnp.full((pad,), TRASH, jnp.int32)]).reshape(NW * JPT, CHUNK)
    xp = jnp.pad(x, ((0, NP - N), (0, 0)))

    degp = _deg_kernel(dstp)
    dega = degp[0].reshape(NP, 1)
    degb = degp[1].reshape(NP, 1)

    y1, dinv = _scale_matmul(xp, W1, dega, degb)
    acc1 = _agg_kernel(y1, srcp, dstp)
    y2 = _mid(acc1[0], acc1[1], y1, dinv, b1.reshape(1, D_HID), W2)
    acc2 = _agg_kernel(y2, srcp, dstp)
    out = _out(acc2[0], acc2[1], y2, dinv, b2.reshape(1, D_HID), Wout,
               bout.reshape(1, 1))
    return out.reshape(NP)[:N]


# speedup vs baseline: 33.9135x; 1.7713x over previous
import functools

import jax
import jax.numpy as jnp
from jax import lax
from jax.experimental import pallas as pl
from jax.experimental.pallas import tpu as pltpu
from jax.experimental.pallas import tpu_sc as plsc

N = 10000
NP = 10240
E = 320000
D_IN = 128
D_HID = 64

NC, NS = 2, 16
NW = NC * NS
CHUNK = 128
JPT = 80
EPT = JPT * CHUNK
EPAD = NW * EPT
TRASH = N
RPT = NP // NS

_MESH = plsc.VectorSubcoreMesh(core_axis_name="c", subcore_axis_name="s")


@functools.partial(
    pl.kernel,
    out_type=jax.ShapeDtypeStruct((NC, NP), jnp.float32),
    mesh=_MESH,
    scratch_types=[
        pltpu.VMEM_SHARED((NP,), jnp.float32),
        pltpu.VMEM((JPT, CHUNK), jnp.int32),
        pltpu.VMEM((CHUNK,), jnp.float32),
        pltpu.VMEM((RPT,), jnp.float32),
    ],
    compiler_params=pltpu.CompilerParams(use_tc_tiling_on_sc=False),
)
def _deg_kernel(dst_hbm, out_hbm, acc, dst_idx, ones_v, zeros_v):
    cid = lax.axis_index("c")
    sid = lax.axis_index("s")
    wid = cid * NS + sid

    for i in range(RPT // 16):
        zeros_v[pl.ds(16 * i, 16)] = jnp.zeros((16,), jnp.float32)
    for i in range(CHUNK // 16):
        ones_v[pl.ds(16 * i, 16)] = jnp.ones((16,), jnp.float32)

    pltpu.sync_copy(dst_hbm.at[pl.ds(wid * JPT, JPT)], dst_idx)
    pltpu.sync_copy(zeros_v, acc.at[pl.ds(sid * RPT, RPT)])
    plsc.subcore_barrier()

    def body(j, carry):
        pltpu.sync_copy(ones_v, acc.at[dst_idx.at[j]], add=True)
        return carry

    lax.fori_loop(0, JPT, body, 0)
    plsc.subcore_barrier()
    pltpu.sync_copy(acc.at[pl.ds(sid * RPT, RPT)],
                    out_hbm.at[cid, pl.ds(sid * RPT, RPT)])


@functools.partial(
    pl.kernel,
    out_type=jax.ShapeDtypeStruct((NC, NP, D_HID), jnp.float32),
    mesh=_MESH,
    scratch_types=[
        pltpu.VMEM_SHARED((NP, D_HID), jnp.float32),
        pltpu.VMEM_SHARED((NP, D_HID), jnp.float32),
        pltpu.VMEM((JPT, CHUNK), jnp.int32),
        pltpu.VMEM((JPT, CHUNK), jnp.int32),
        pltpu.VMEM((CHUNK, D_HID), jnp.float32),
        pltpu.VMEM((CHUNK, D_HID), jnp.float32),
        pltpu.SemaphoreType.DMA,
        pltpu.SemaphoreType.DMA,
    ],
    compiler_params=pltpu.CompilerParams(use_tc_tiling_on_sc=False),
)
def _agg_kernel(y_hbm, src_hbm, dst_hbm, out_hbm, acc, ytab, src_idx, dst_idx,
                rows0, rows1, sem0, sem1):
    cid = lax.axis_index("c")
    sid = lax.axis_index("s")
    wid = cid * NS + sid

    pltpu.sync_copy(src_hbm.at[pl.ds(wid * JPT, JPT)], src_idx)
    pltpu.sync_copy(dst_hbm.at[pl.ds(wid * JPT, JPT)], dst_idx)
    r0 = sid * RPT
    pltpu.sync_copy(y_hbm.at[pl.ds(r0, RPT)], acc.at[pl.ds(r0, RPT)])
    pltpu.sync_copy(y_hbm.at[pl.ds(r0, RPT)], ytab.at[pl.ds(r0, RPT)])
    plsc.subcore_barrier()

    pltpu.async_copy(ytab.at[src_idx.at[0]], rows0, sem0)
    pltpu.async_copy(ytab.at[src_idx.at[1]], rows1, sem1)

    def half(j, rows, sem):
        pltpu.make_async_copy(ytab.at[src_idx.at[j]], rows, sem).wait()
        pltpu.sync_copy(rows, acc.at[dst_idx.at[j]], add=True)

        @pl.when(j + 2 < JPT)
        def _():
            pltpu.async_copy(ytab.at[src_idx.at[j + 2]], rows, sem)

    def body(jj, carry):
        half(2 * jj, rows0, sem0)
        half(2 * jj + 1, rows1, sem1)
        return carry

    lax.fori_loop(0, JPT // 2, body, 0)
    plsc.subcore_barrier()
    pltpu.sync_copy(acc.at[pl.ds(r0, RPT)], out_hbm.at[cid, pl.ds(r0, RPT)])


_BS = 1024


def _tc_scale_matmul(x_ref, w_ref, da_ref, db_ref, y_ref, dinv_ref):
    deg = 1.0 + da_ref[...] + db_ref[...]
    dinv = lax.rsqrt(deg)
    y_ref[...] = dinv * jnp.dot(x_ref[...], w_ref[...],
                                preferred_element_type=jnp.float32)
    dinv_ref[...] = dinv


def _tc_mid(a_ref, b_ref, y_ref, dinv_ref, bias_ref, w_ref, out_ref):
    s = a_ref[...] + b_ref[...] - y_ref[...]
    h = jnp.maximum(dinv_ref[...] * s + bias_ref[...], 0.0)
    out_ref[...] = dinv_ref[...] * jnp.dot(h, w_ref[...],
                                           preferred_element_type=jnp.float32)


def _tc_out(a_ref, b_ref, y_ref, dinv_ref, bias_ref, w_ref, bout_ref, out_ref):
    s = a_ref[...] + b_ref[...] - y_ref[...]
    h = jnp.maximum(dinv_ref[...] * s + bias_ref[...], 0.0)
    out_ref[...] = jnp.dot(h, w_ref[...],
                           preferred_element_type=jnp.float32) + bout_ref[...]


def _row_spec(d):
    return pl.BlockSpec((_BS, d), lambda i: (i, 0))


def _full_spec(r, c):
    return pl.BlockSpec((r, c), lambda i: (0, 0))


_scale_matmul = pl.pallas_call(
    _tc_scale_matmul,
    grid=(NP // _BS,),
    in_specs=[_row_spec(D_IN), _full_spec(D_IN, D_HID), _row_spec(1),
              _row_spec(1)],
    out_specs=[_row_spec(D_HID), _row_spec(1)],
    out_shape=[jax.ShapeDtypeStruct((NP, D_HID), jnp.float32),
               jax.ShapeDtypeStruct((NP, 1), jnp.float32)],
)

_mid = pl.pallas_call(
    _tc_mid,
    grid=(NP // _BS,),
    in_specs=[_row_spec(D_HID), _row_spec(D_HID), _row_spec(D_HID),
              _row_spec(1), _full_spec(1, D_HID), _full_spec(D_HID, D_HID)],
    out_specs=_row_spec(D_HID),
    out_shape=jax.ShapeDtypeStruct((NP, D_HID), jnp.float32),
)

_out = pl.pallas_call(
    _tc_out,
    grid=(NP // _BS,),
    in_specs=[_row_spec(D_HID), _row_spec(D_HID), _row_spec(D_HID),
              _row_spec(1), _full_spec(1, D_HID), _full_spec(D_HID, 1),
              _full_spec(1, 1)],
    out_specs=_row_spec(1),
    out_shape=jax.ShapeDtypeStruct((NP, 1), jnp.float32),
)


def kernel(x, edge_index, W1, b1, W2, b2, Wout, bout):
    src = edge_index[0]
    dst = edge_index[1]
    pad = EPAD - E
    srcp = jnp.concatenate(
        [src, jnp.zeros((pad,), jnp.int32)]).reshape(NW * JPT, CHUNK)
    dstp = jnp.concatenate(
        [dst, jnp.full((pad,), TRASH, jnp.int32)]).reshape(NW * JPT, CHUNK)
    xp = jnp.pad(x, ((0, NP - N), (0, 0)))

    degp = _deg_kernel(dstp)
    dega = degp[0].reshape(NP, 1)
    degb = degp[1].reshape(NP, 1)

    y1, dinv = _scale_matmul(xp, W1, dega, degb)
    acc1 = _agg_kernel(y1, srcp, dstp)
    y2 = _mid(acc1[0], acc1[1], y1, dinv, b1.reshape(1, D_HID), W2)
    acc2 = _agg_kernel(y2, srcp, dstp)
    out = _out(acc2[0], acc2[1], y2, dinv, b2.reshape(1, D_HID), Wout,
               bout.reshape(1, 1))
    return out.reshape(NP)[:N]


# speedup vs baseline: 35.7524x; 1.0542x over previous
import functools

import jax
import jax.numpy as jnp
from jax import lax
from jax.experimental import pallas as pl
from jax.experimental.pallas import tpu as pltpu
from jax.experimental.pallas import tpu_sc as plsc

N = 10000
NP = 10240
E = 320000
D_IN = 128
D_HID = 64

NC, NS = 2, 16
NW = NC * NS
CHUNK = 128
JPT = 80
EPT = JPT * CHUNK
EPAD = NW * EPT
TRASH = N
RPT = NP // NS

_MESH = plsc.VectorSubcoreMesh(core_axis_name="c", subcore_axis_name="s")


@functools.partial(
    pl.kernel,
    out_type=jax.ShapeDtypeStruct((NC, NP), jnp.float32),
    mesh=_MESH,
    scratch_types=[
        pltpu.VMEM_SHARED((NP,), jnp.float32),
        pltpu.VMEM((JPT, CHUNK), jnp.int32),
        pltpu.VMEM((CHUNK,), jnp.float32),
        pltpu.VMEM((RPT,), jnp.float32),
    ],
    compiler_params=pltpu.CompilerParams(use_tc_tiling_on_sc=False),
)
def _deg_kernel(dst_hbm, out_hbm, acc, dst_idx, ones_v, zeros_v):
    cid = lax.axis_index("c")
    sid = lax.axis_index("s")
    wid = cid * NS + sid

    for i in range(RPT // 16):
        zeros_v[pl.ds(16 * i, 16)] = jnp.zeros((16,), jnp.float32)
    for i in range(CHUNK // 16):
        ones_v[pl.ds(16 * i, 16)] = jnp.ones((16,), jnp.float32)

    pltpu.sync_copy(dst_hbm.at[pl.ds(wid * JPT, JPT)], dst_idx)
    pltpu.sync_copy(zeros_v, acc.at[pl.ds(sid * RPT, RPT)])
    plsc.subcore_barrier()

    def body(j, carry):
        pltpu.sync_copy(ones_v, acc.at[dst_idx.at[j]], add=True)
        return carry

    lax.fori_loop(0, JPT, body, 0)
    plsc.subcore_barrier()
    pltpu.sync_copy(acc.at[pl.ds(sid * RPT, RPT)],
                    out_hbm.at[cid, pl.ds(sid * RPT, RPT)])


@functools.partial(
    pl.kernel,
    out_type=jax.ShapeDtypeStruct((NC, NP, D_HID), jnp.float32),
    mesh=_MESH,
    scratch_types=[
        pltpu.VMEM_SHARED((NP, D_HID), jnp.float32),
        pltpu.VMEM_SHARED((NP, D_HID), jnp.float32),
        pltpu.VMEM((JPT, CHUNK), jnp.int32),
        pltpu.VMEM((JPT, CHUNK), jnp.int32),
        pltpu.VMEM((CHUNK, D_HID), jnp.float32),
        pltpu.VMEM((CHUNK, D_HID), jnp.float32),
        pltpu.SemaphoreType.DMA,
        pltpu.SemaphoreType.DMA,
    ],
    compiler_params=pltpu.CompilerParams(use_tc_tiling_on_sc=False),
)
def _agg_kernel(y_hbm, src_hbm, dst_hbm, out_hbm, acc, ytab, src_idx, dst_idx,
                rows0, rows1, sem0, sem1):
    cid = lax.axis_index("c")
    sid = lax.axis_index("s")
    wid = cid * NS + sid

    pltpu.sync_copy(src_hbm.at[pl.ds(wid * JPT, JPT)], src_idx)
    pltpu.sync_copy(dst_hbm.at[pl.ds(wid * JPT, JPT)], dst_idx)
    r0 = sid * RPT
    pltpu.sync_copy(y_hbm.at[pl.ds(r0, RPT)], acc.at[pl.ds(r0, RPT)])
    pltpu.sync_copy(y_hbm.at[pl.ds(r0, RPT)], ytab.at[pl.ds(r0, RPT)])
    plsc.subcore_barrier()

    pltpu.async_copy(ytab.at[src_idx.at[0]], rows0, sem0)
    pltpu.async_copy(ytab.at[src_idx.at[1]], rows1, sem1)

    def half(j, rows, sem):
        pltpu.make_async_copy(ytab.at[src_idx.at[j]], rows, sem).wait()
        pltpu.sync_copy(rows, acc.at[dst_idx.at[j]], add=True)

        @pl.when(j + 2 < JPT)
        def _():
            pltpu.async_copy(ytab.at[src_idx.at[j + 2]], rows, sem)

    def body(jj, carry):
        half(2 * jj, rows0, sem0)
        half(2 * jj + 1, rows1, sem1)
        return carry

    lax.fori_loop(0, JPT // 2, body, 0)
    plsc.subcore_barrier()
    pltpu.sync_copy(acc.at[pl.ds(r0, RPT)], out_hbm.at[cid, pl.ds(r0, RPT)])


_BS = 1024


def _tc_scale_matmul(x_ref, w_ref, deg_ref, y_ref, dinv_ref):
    deg = 1.0 + deg_ref[0] + deg_ref[1]
    dinv = lax.rsqrt(deg)
    y_ref[...] = dinv * jnp.dot(x_ref[...], w_ref[...],
                                preferred_element_type=jnp.float32)
    dinv_ref[...] = dinv


def _tc_mid(acc_ref, y_ref, dinv_ref, bias_ref, w_ref, out_ref):
    s = acc_ref[0] + acc_ref[1] - y_ref[...]
    h = jnp.maximum(dinv_ref[...] * s + bias_ref[...], 0.0)
    out_ref[...] = dinv_ref[...] * jnp.dot(h, w_ref[...],
                                           preferred_element_type=jnp.float32)


def _tc_out(acc_ref, y_ref, dinv_ref, bias_ref, w_ref, bout_ref, out_ref):
    s = acc_ref[0] + acc_ref[1] - y_ref[...]
    h = jnp.maximum(dinv_ref[...] * s + bias_ref[...], 0.0)
    out_ref[...] = jnp.dot(h, w_ref[...],
                           preferred_element_type=jnp.float32) + bout_ref[...]


_scale_matmul = pl.pallas_call(
    _tc_scale_matmul,
    out_shape=[jax.ShapeDtypeStruct((NP, D_HID), jnp.float32),
               jax.ShapeDtypeStruct((NP, 1), jnp.float32)],
)

_mid = pl.pallas_call(
    _tc_mid,
    out_shape=jax.ShapeDtypeStruct((NP, D_HID), jnp.float32),
)

_out = pl.pallas_call(
    _tc_out,
    out_shape=jax.ShapeDtypeStruct((NP, 1), jnp.float32),
)


def kernel(x, edge_index, W1, b1, W2, b2, Wout, bout):
    src = edge_index[0]
    dst = edge_index[1]
    pad = EPAD - E
    srcp = jnp.concatenate(
        [src, jnp.zeros((pad,), jnp.int32)]).reshape(NW * JPT, CHUNK)
    dstp = jnp.concatenate(
        [dst, jnp.full((pad,), TRASH, jnp.int32)]).reshape(NW * JPT, CHUNK)
    xp = jnp.pad(x, ((0, NP - N), (0, 0)))

    degp = _deg_kernel(dstp).reshape(NC, NP, 1)

    y1, dinv = _scale_matmul(xp, W1, degp)
    acc1 = _agg_kernel(y1, srcp, dstp)
    y2 = _mid(acc1, y1, dinv, b1.reshape(1, D_HID), W2)
    acc2 = _agg_kernel(y2, srcp, dstp)
    out = _out(acc2, y2, dinv, b2.reshape(1, D_HID), Wout, bout.reshape(1, 1))
    return out.reshape(NP)[:N]


# speedup vs baseline: 40.0438x; 1.1200x over previous
import functools

import jax
import jax.numpy as jnp
from jax import lax
from jax.experimental import pallas as pl
from jax.experimental.pallas import tpu as pltpu
from jax.experimental.pallas import tpu_sc as plsc

N = 10000
NP = 10240
E = 320000
D_IN = 128
D_HID = 64

NC, NS = 2, 16
NW = NC * NS
CHUNK = 128
JPT = 80
EPT = JPT * CHUNK
EPAD = NW * EPT
TRASH = N
RPT = NP // NS

_MESH = plsc.VectorSubcoreMesh(core_axis_name="c", subcore_axis_name="s")


@functools.partial(
    pl.kernel,
    out_type=jax.ShapeDtypeStruct((NC, NP), jnp.float32),
    mesh=_MESH,
    scratch_types=[
        pltpu.VMEM_SHARED((NP,), jnp.float32),
        pltpu.VMEM((JPT, CHUNK), jnp.int32),
        pltpu.VMEM((CHUNK,), jnp.float32),
        pltpu.VMEM((RPT,), jnp.float32),
        pltpu.SemaphoreType.DMA,
    ],
    compiler_params=pltpu.CompilerParams(use_tc_tiling_on_sc=False),
)
def _deg_kernel(ei_hbm, out_hbm, acc, dst_idx, ones_v, zeros_v, sem):
    cid = lax.axis_index("c")
    sid = lax.axis_index("s")
    wid = cid * NS + sid

    for i in range(RPT // 16):
        zeros_v[pl.ds(16 * i, 16)] = jnp.zeros((16,), jnp.float32)
    for i in range(CHUNK // 16):
        ones_v[pl.ds(16 * i, 16)] = jnp.ones((16,), jnp.float32)

    pltpu.sync_copy(ei_hbm.at[1, pl.ds(wid * JPT, JPT)], dst_idx)
    pltpu.sync_copy(zeros_v, acc.at[pl.ds(sid * RPT, RPT)])
    plsc.subcore_barrier()

    def fire(j, carry):
        pltpu.async_copy(ones_v, acc.at[dst_idx.at[j]], sem, add=True)
        return carry

    lax.fori_loop(0, JPT, fire, 0)

    def drain(j, carry):
        pltpu.make_async_copy(ones_v, acc.at[dst_idx.at[j]], sem).wait()
        return carry

    lax.fori_loop(0, JPT, drain, 0)
    plsc.subcore_barrier()
    pltpu.sync_copy(acc.at[pl.ds(sid * RPT, RPT)],
                    out_hbm.at[cid, pl.ds(sid * RPT, RPT)])


@functools.partial(
    pl.kernel,
    out_type=jax.ShapeDtypeStruct((NC, NP, D_HID), jnp.float32),
    mesh=_MESH,
    scratch_types=[
        pltpu.VMEM_SHARED((NP, D_HID), jnp.float32),
        pltpu.VMEM_SHARED((NP, D_HID), jnp.float32),
        pltpu.VMEM((JPT // 2, CHUNK), jnp.int32),
        pltpu.VMEM((JPT // 2, CHUNK), jnp.int32),
        pltpu.VMEM((CHUNK, D_HID), jnp.float32),
        pltpu.VMEM((CHUNK, D_HID), jnp.float32),
        pltpu.VMEM((CHUNK, D_HID), jnp.float32),
        pltpu.VMEM((CHUNK, D_HID), jnp.float32),
        pltpu.SemaphoreType.DMA,
        pltpu.SemaphoreType.DMA,
        pltpu.SemaphoreType.DMA,
        pltpu.SemaphoreType.DMA,
    ],
    compiler_params=pltpu.CompilerParams(use_tc_tiling_on_sc=False),
)
def _agg_kernel(y_hbm, ei_hbm, out_hbm, acc, ytab, src_idx, dst_idx,
                rows0, rows1, rows2, rows3, sem0, sem1, sem2, sem3):
    cid = lax.axis_index("c")
    sid = lax.axis_index("s")
    wid = cid * NS + sid
    rows = (rows0, rows1, rows2, rows3)
    sems = (sem0, sem1, sem2, sem3)

    r0 = sid * RPT
    pltpu.sync_copy(y_hbm.at[pl.ds(r0, RPT)], acc.at[pl.ds(r0, RPT)])
    pltpu.sync_copy(y_hbm.at[pl.ds(r0, RPT)], ytab.at[pl.ds(r0, RPT)])
    plsc.subcore_barrier()

    def gather(j, k):
        pltpu.async_copy(ytab.at[src_idx.at[j]], rows[k], sems[k])

    def gather_wait(j, k):
        pltpu.make_async_copy(ytab.at[src_idx.at[j]], rows[k], sems[k]).wait()

    def scatter(j, k):
        pltpu.async_copy(rows[k], acc.at[dst_idx.at[j]], sems[k], add=True)

    def scatter_wait(j, k):
        pltpu.make_async_copy(rows[k], acc.at[dst_idx.at[j]], sems[k]).wait()

    JH = JPT // 2
    for phase in range(2):
        base_c = wid * JPT + phase * JH
        pltpu.sync_copy(ei_hbm.at[0, pl.ds(base_c, JH)], src_idx)
        pltpu.sync_copy(ei_hbm.at[1, pl.ds(base_c, JH)], dst_idx)

        gather(0, 0)
        gather(1, 1)
        gather_wait(0, 0)
        scatter(0, 0)
        gather(2, 2)
        gather_wait(1, 1)
        scatter(1, 1)
        gather(3, 3)

        def body(jj, carry):
            base = 4 * jj + 2
            for t in range(4):
                j = base + t
                k = (2 + t) % 4
                kn = t
                gather_wait(j, k)
                scatter(j, k)
                scatter_wait(j - 2, kn)
                gather(j + 2, kn)
            return carry

        lax.fori_loop(0, (JH - 4) // 4, body, 0)
        gather_wait(JH - 2, 2)
        scatter(JH - 2, 2)
        scatter_wait(JH - 4, 0)
        gather_wait(JH - 1, 3)
        scatter(JH - 1, 3)
        scatter_wait(JH - 3, 1)
        scatter_wait(JH - 2, 2)
        scatter_wait(JH - 1, 3)
    plsc.subcore_barrier()
    pltpu.sync_copy(acc.at[pl.ds(r0, RPT)], out_hbm.at[cid, pl.ds(r0, RPT)])


_BS = 1024


def _tc_scale_matmul(x_ref, w_ref, deg_ref, y_ref, dinv_ref):
    deg = 1.0 + deg_ref[0] + deg_ref[1]
    dinv = lax.rsqrt(deg)
    y_ref[...] = dinv * jnp.dot(x_ref[...], w_ref[...],
                                preferred_element_type=jnp.float32)
    dinv_ref[...] = dinv


def _tc_mid(acc_ref, y_ref, dinv_ref, bias_ref, w_ref, out_ref):
    s = acc_ref[0] + acc_ref[1] - y_ref[...]
    h = jnp.maximum(dinv_ref[...] * s + bias_ref[...], 0.0)
    out_ref[...] = dinv_ref[...] * jnp.dot(h, w_ref[...],
                                           preferred_element_type=jnp.float32)


def _tc_out(acc_ref, y_ref, dinv_ref, bias_ref, w_ref, bout_ref, out_ref):
    s = acc_ref[0] + acc_ref[1] - y_ref[...]
    h = jnp.maximum(dinv_ref[...] * s + bias_ref[...], 0.0)
    out_ref[...] = jnp.dot(h, w_ref[...],
                           preferred_element_type=jnp.float32) + bout_ref[...]


_scale_matmul = pl.pallas_call(
    _tc_scale_matmul,
    out_shape=[jax.ShapeDtypeStruct((NP, D_HID), jnp.float32),
               jax.ShapeDtypeStruct((NP, 1), jnp.float32)],
)

_mid = pl.pallas_call(
    _tc_mid,
    out_shape=jax.ShapeDtypeStruct((NP, D_HID), jnp.float32),
)

_out = pl.pallas_call(
    _tc_out,
    out_shape=jax.ShapeDtypeStruct((NP, 1), jnp.float32),
)


def kernel(x, edge_index, W1, b1, W2, b2, Wout, bout):
    pad = EPAD - E
    eip = jnp.pad(edge_index, ((0, 0), (0, pad)),
                  constant_values=TRASH).reshape(2, NW * JPT, CHUNK)
    xp = jnp.pad(x, ((0, NP - N), (0, 0)))

    degp = _deg_kernel(eip).reshape(NC, NP, 1)

    y1, dinv = _scale_matmul(xp, W1, degp)
    acc1 = _agg_kernel(y1, eip)
    y2 = _mid(acc1, y1, dinv, b1.reshape(1, D_HID), W2)
    acc2 = _agg_kernel(y2, eip)
    out = _out(acc2, y2, dinv, b2.reshape(1, D_HID), Wout, bout.reshape(1, 1))
    return out.reshape(NP)[:N]
